# Initial kernel scaffold; baseline (speedup 1.0000x reference)
import jax
import jax.numpy as jnp
from jax.experimental import pallas as pl


def kernel(x, edge_index, W1l, W1r, b1, W2l, W2r, b2, W3l, W3r, b3, Wreg, breg, Wcls, bcls):
    raise NotImplementedError("write your pallas kernel here")


# speedup vs baseline: 5.9062x; 5.9062x over previous
import functools

import jax
import jax.numpy as jnp
from jax import lax
from jax.experimental import pallas as pl
from jax.experimental.pallas import tpu as pltpu
from jax.experimental.pallas import tpu_sc as plsc

_CHUNK = 128
_PAD_COLS = 16


def _node_rows(n):
    return ((n + 1 + 2047) // 2048) * 2048


def _sc_segsum(y, src, dst, n):
    R, D = y.shape
    E = src.shape[0]
    mesh = plsc.VectorSubcoreMesh(core_axis_name="c", subcore_axis_name="s")
    NC, NS = mesh.num_cores, mesh.num_subcores
    NW = NC * NS
    cpt = -(-E // (NW * _CHUNK))
    E_pad = NW * cpt * _CHUNK
    src = jnp.concatenate([src, jnp.zeros((E_pad - E,), jnp.int32)])
    dst = jnp.concatenate([dst, jnp.full((E_pad - E,), n, jnp.int32)])
    zeros = jnp.zeros((R, D), jnp.float32)
    rpt = R // NS

    @functools.partial(
        pl.kernel,
        out_type=jax.ShapeDtypeStruct((NC, R, D), jnp.float32),
        mesh=mesh,
        scratch_types=[
            pltpu.VMEM((_CHUNK,), jnp.int32),
            pltpu.VMEM((_CHUNK,), jnp.int32),
            pltpu.VMEM((_CHUNK, D), jnp.float32),
            pltpu.VMEM_SHARED((R, D), jnp.float32),
            pltpu.SemaphoreType.DMA,
        ],
        compiler_params=pltpu.CompilerParams(use_tc_tiling_on_sc=False),
    )
    def seg_kernel(y_hbm, src_hbm, dst_hbm, z_hbm, out_hbm,
                   sidx, didx, rows, agg, sem):
        cid = lax.axis_index("c")
        sid = lax.axis_index("s")
        wid = cid * NS + sid
        base = sid * rpt
        pltpu.sync_copy(z_hbm.at[pl.ds(base, rpt)], agg.at[pl.ds(base, rpt)])
        plsc.subcore_barrier()
        ebase = wid * (cpt * _CHUNK)

        def body(i, carry):
            off = ebase + i * _CHUNK
            pltpu.sync_copy(src_hbm.at[pl.ds(off, _CHUNK)], sidx)
            pltpu.sync_copy(dst_hbm.at[pl.ds(off, _CHUNK)], didx)
            pltpu.async_copy(y_hbm.at[sidx], rows, sem).wait()
            pltpu.sync_copy(rows, agg.at[didx], add=True)
            return carry

        lax.fori_loop(0, cpt, body, 0)
        plsc.subcore_barrier()
        pltpu.sync_copy(agg.at[pl.ds(base, rpt)],
                        out_hbm.at[cid, pl.ds(base, rpt)])

    return seg_kernel(y, src, dst, zeros)


def _tc_proj_first(x, wl, ones_bias, wr, br):
    R = x.shape[0]
    D = wl.shape[1]

    def body(x_ref, wl_ref, ob_ref, wr_ref, br_ref, y_ref, yr_ref):
        xv = x_ref[...]
        y_ref[...] = (jnp.dot(xv, wl_ref[...],
                              preferred_element_type=jnp.float32)
                      + ob_ref[...][None, :])
        yr_ref[...] = (jnp.dot(xv, wr_ref[...],
                               preferred_element_type=jnp.float32)
                       + br_ref[...][None, :])

    return pl.pallas_call(
        body,
        out_shape=[jax.ShapeDtypeStruct((R, D), jnp.float32),
                   jax.ShapeDtypeStruct((R, D), jnp.float32)],
    )(x, wl, ones_bias, wr, br)


def _tc_mean_proj(p, yr, sel, wl, wr, br):
    _, R, _ = p.shape
    D2 = wl.shape[1]

    def body(p_ref, yr_ref, sel_ref, wl_ref, wr_ref, br_ref,
             y_ref, y2r_ref, inv_ref):
        agg = p_ref[0] + p_ref[1]
        cnt = jnp.dot(agg, sel_ref[...],
                      preferred_element_type=jnp.float32)
        inv = 1.0 / jnp.maximum(cnt, 1.0)
        h = jnp.maximum(agg * inv + yr_ref[...], 0.0)
        y_ref[...] = jnp.dot(h, wl_ref[...],
                             preferred_element_type=jnp.float32)
        y2r_ref[...] = (jnp.dot(h, wr_ref[...],
                                preferred_element_type=jnp.float32)
                        + br_ref[...][None, :])
        inv_ref[...] = inv

    return pl.pallas_call(
        body,
        out_shape=[jax.ShapeDtypeStruct((R, D2), jnp.float32),
                   jax.ShapeDtypeStruct((R, D2), jnp.float32),
                   jax.ShapeDtypeStruct((R, 1), jnp.float32)],
    )(p, yr, sel, wl, wr, br)


def _tc_mid(p, yr, inv, wl, wr, br):
    _, R, _ = p.shape
    D2 = wl.shape[1]

    def body(p_ref, yr_ref, inv_ref, wl_ref, wr_ref, br_ref, y_ref, yr2_ref):
        agg = p_ref[0] + p_ref[1]
        h = jnp.maximum(agg * inv_ref[...] + yr_ref[...], 0.0)
        y_ref[...] = jnp.dot(h, wl_ref[...],
                             preferred_element_type=jnp.float32)
        yr2_ref[...] = (jnp.dot(h, wr_ref[...],
                                preferred_element_type=jnp.float32)
                        + br_ref[...][None, :])

    return pl.pallas_call(
        body,
        out_shape=[jax.ShapeDtypeStruct((R, D2), jnp.float32),
                   jax.ShapeDtypeStruct((R, D2), jnp.float32)],
    )(p, yr, inv, wl, wr, br)


def _tc_final(p, yr, inv, w_head, b_head):
    _, R, _ = p.shape

    def body(p_ref, yr_ref, inv_ref, wh_ref, bh_ref, o_ref):
        agg = p_ref[0] + p_ref[1]
        h = jnp.maximum(agg * inv_ref[...] + yr_ref[...], 0.0)
        o_ref[...] = (jnp.dot(h, wh_ref[...],
                              preferred_element_type=jnp.float32)
                      + bh_ref[...][None, :])

    return pl.pallas_call(
        body,
        out_shape=jax.ShapeDtypeStruct((R, 2), jnp.float32),
    )(p, yr, inv, w_head, b_head)


def kernel(x, edge_index, W1l, W1r, b1, W2l, W2r, b2, W3l, W3r, b3,
           Wreg, breg, Wcls, bcls):
    n, d_in = x.shape
    R = _node_rows(n)
    d1 = W1l.shape[1]
    d1p = d1 + _PAD_COLS

    x_pad = jnp.zeros((R, d_in), jnp.float32).at[:n].set(x)
    src = edge_index[0].astype(jnp.int32)
    dst = edge_index[1].astype(jnp.int32)

    W1l_p = jnp.pad(W1l, ((0, 0), (0, _PAD_COLS)))
    ones_bias = jnp.concatenate(
        [jnp.zeros((d1,), jnp.float32), jnp.ones((_PAD_COLS,), jnp.float32)])
    W1r_p = jnp.pad(W1r, ((0, 0), (0, _PAD_COLS)))
    b1_p = jnp.pad(b1, (0, _PAD_COLS))
    sel = jnp.zeros((d1p, 1), jnp.float32).at[d1, 0].set(1.0)
    W2l_p = jnp.pad(W2l, ((0, _PAD_COLS), (0, 0)))
    W2r_p = jnp.pad(W2r, ((0, _PAD_COLS), (0, 0)))

    Y1, y1r = _tc_proj_first(x_pad, W1l_p, ones_bias, W1r_p, b1_p)
    p1 = _sc_segsum(Y1, src, dst, n)
    Y2, y2r, inv = _tc_mean_proj(p1, y1r, sel, W2l_p, W2r_p, b2)
    p2 = _sc_segsum(Y2, src, dst, n)
    Y3, y3r = _tc_mid(p2, y2r, inv, W3l, W3r, b3)
    p3 = _sc_segsum(Y3, src, dst, n)

    w_head = jnp.concatenate([Wreg, Wcls], axis=1)
    b_head = jnp.concatenate([breg, bcls])
    out = _tc_final(p3, y3r, inv, w_head, b_head)
    return out[:n, 0], out[:n, 1]


# speedup vs baseline: 6.5058x; 1.1015x over previous
import functools

import jax
import jax.numpy as jnp
from jax import lax
from jax.experimental import pallas as pl
from jax.experimental.pallas import tpu as pltpu
from jax.experimental.pallas import tpu_sc as plsc

_CHUNK = 128
_PAD_COLS = 16


def _node_rows(n):
    return ((n + 1 + 2047) // 2048) * 2048


def _sc_segsum(y, src, dst, n):
    R, D = y.shape
    E = src.shape[0]
    mesh = plsc.VectorSubcoreMesh(core_axis_name="c", subcore_axis_name="s")
    NC, NS = mesh.num_cores, mesh.num_subcores
    NW = NC * NS
    K = 4
    cpt = -(-E // (NW * _CHUNK * K)) * K
    E_pad = NW * cpt * _CHUNK
    G = cpt // K
    src = jnp.concatenate([src, jnp.zeros((E_pad - E,), jnp.int32)])
    dst = jnp.concatenate([dst, jnp.full((E_pad - E,), n, jnp.int32)])
    src_r = src.reshape(NW, cpt, _CHUNK)
    dst_r = dst.reshape(NW, cpt, _CHUNK)
    zeros = jnp.zeros((R, D), jnp.float32)
    rpt = R // NS

    @functools.partial(
        pl.kernel,
        out_type=jax.ShapeDtypeStruct((NC, R, D), jnp.float32),
        mesh=mesh,
        scratch_types=[
            pltpu.VMEM((cpt, _CHUNK), jnp.int32),
            pltpu.VMEM((cpt, _CHUNK), jnp.int32),
            pltpu.VMEM((K, _CHUNK, D), jnp.float32),
            pltpu.VMEM_SHARED((R, D), jnp.float32),
            pltpu.SemaphoreType.DMA,
        ],
        compiler_params=pltpu.CompilerParams(use_tc_tiling_on_sc=False),
    )
    def seg_kernel(y_hbm, src_hbm, dst_hbm, z_hbm, out_hbm,
                   sidx, didx, rows, agg, sem):
        cid = lax.axis_index("c")
        sid = lax.axis_index("s")
        wid = cid * NS + sid
        base = sid * rpt
        pltpu.sync_copy(src_hbm.at[wid], sidx)
        pltpu.sync_copy(dst_hbm.at[wid], didx)
        pltpu.sync_copy(z_hbm.at[pl.ds(base, rpt)], agg.at[pl.ds(base, rpt)])
        plsc.subcore_barrier()

        def body(g, carry):
            c0 = g * K
            gathers = [
                pltpu.async_copy(y_hbm.at[sidx.at[c0 + b]], rows.at[b], sem)
                for b in range(K)
            ]
            for d in gathers:
                d.wait()
            for b in range(K):
                pltpu.sync_copy(rows.at[b], agg.at[didx.at[c0 + b]], add=True)
            return carry

        lax.fori_loop(0, G, body, 0)
        plsc.subcore_barrier()
        pltpu.sync_copy(agg.at[pl.ds(base, rpt)],
                        out_hbm.at[cid, pl.ds(base, rpt)])

    return seg_kernel(y, src_r, dst_r, zeros)


def _tc_proj_first(x, wl, ones_bias, wr, br):
    R = x.shape[0]
    D = wl.shape[1]

    def body(x_ref, wl_ref, ob_ref, wr_ref, br_ref, y_ref, yr_ref):
        xv = x_ref[...]
        y_ref[...] = (jnp.dot(xv, wl_ref[...],
                              preferred_element_type=jnp.float32)
                      + ob_ref[...][None, :])
        yr_ref[...] = (jnp.dot(xv, wr_ref[...],
                               preferred_element_type=jnp.float32)
                       + br_ref[...][None, :])

    return pl.pallas_call(
        body,
        out_shape=[jax.ShapeDtypeStruct((R, D), jnp.float32),
                   jax.ShapeDtypeStruct((R, D), jnp.float32)],
    )(x, wl, ones_bias, wr, br)


def _tc_mean_proj(p, yr, sel, wl, wr, br):
    _, R, _ = p.shape
    D2 = wl.shape[1]

    def body(p_ref, yr_ref, sel_ref, wl_ref, wr_ref, br_ref,
             y_ref, y2r_ref, inv_ref):
        agg = p_ref[0] + p_ref[1]
        cnt = jnp.dot(agg, sel_ref[...],
                      preferred_element_type=jnp.float32)
        inv = 1.0 / jnp.maximum(cnt, 1.0)
        h = jnp.maximum(agg * inv + yr_ref[...], 0.0)
        y_ref[...] = jnp.dot(h, wl_ref[...],
                             preferred_element_type=jnp.float32)
        y2r_ref[...] = (jnp.dot(h, wr_ref[...],
                                preferred_element_type=jnp.float32)
                        + br_ref[...][None, :])
        inv_ref[...] = inv

    return pl.pallas_call(
        body,
        out_shape=[jax.ShapeDtypeStruct((R, D2), jnp.float32),
                   jax.ShapeDtypeStruct((R, D2), jnp.float32),
                   jax.ShapeDtypeStruct((R, 1), jnp.float32)],
    )(p, yr, sel, wl, wr, br)


def _tc_mid(p, yr, inv, wl, wr, br):
    _, R, _ = p.shape
    D2 = wl.shape[1]

    def body(p_ref, yr_ref, inv_ref, wl_ref, wr_ref, br_ref, y_ref, yr2_ref):
        agg = p_ref[0] + p_ref[1]
        h = jnp.maximum(agg * inv_ref[...] + yr_ref[...], 0.0)
        y_ref[...] = jnp.dot(h, wl_ref[...],
                             preferred_element_type=jnp.float32)
        yr2_ref[...] = (jnp.dot(h, wr_ref[...],
                                preferred_element_type=jnp.float32)
                        + br_ref[...][None, :])

    return pl.pallas_call(
        body,
        out_shape=[jax.ShapeDtypeStruct((R, D2), jnp.float32),
                   jax.ShapeDtypeStruct((R, D2), jnp.float32)],
    )(p, yr, inv, wl, wr, br)


def _tc_final(p, yr, inv, w_head, b_head):
    _, R, _ = p.shape

    def body(p_ref, yr_ref, inv_ref, wh_ref, bh_ref, o_ref):
        agg = p_ref[0] + p_ref[1]
        h = jnp.maximum(agg * inv_ref[...] + yr_ref[...], 0.0)
        o_ref[...] = (jnp.dot(h, wh_ref[...],
                              preferred_element_type=jnp.float32)
                      + bh_ref[...][None, :])

    return pl.pallas_call(
        body,
        out_shape=jax.ShapeDtypeStruct((R, 2), jnp.float32),
    )(p, yr, inv, w_head, b_head)


def kernel(x, edge_index, W1l, W1r, b1, W2l, W2r, b2, W3l, W3r, b3,
           Wreg, breg, Wcls, bcls):
    n, d_in = x.shape
    R = _node_rows(n)
    d1 = W1l.shape[1]
    d1p = d1 + _PAD_COLS

    x_pad = jnp.zeros((R, d_in), jnp.float32).at[:n].set(x)
    src = edge_index[0].astype(jnp.int32)
    dst = edge_index[1].astype(jnp.int32)

    W1l_p = jnp.pad(W1l, ((0, 0), (0, _PAD_COLS)))
    ones_bias = jnp.concatenate(
        [jnp.zeros((d1,), jnp.float32), jnp.ones((_PAD_COLS,), jnp.float32)])
    W1r_p = jnp.pad(W1r, ((0, 0), (0, _PAD_COLS)))
    b1_p = jnp.pad(b1, (0, _PAD_COLS))
    sel = jnp.zeros((d1p, 1), jnp.float32).at[d1, 0].set(1.0)
    W2l_p = jnp.pad(W2l, ((0, _PAD_COLS), (0, 0)))
    W2r_p = jnp.pad(W2r, ((0, _PAD_COLS), (0, 0)))

    Y1, y1r = _tc_proj_first(x_pad, W1l_p, ones_bias, W1r_p, b1_p)
    p1 = _sc_segsum(Y1, src, dst, n)
    Y2, y2r, inv = _tc_mean_proj(p1, y1r, sel, W2l_p, W2r_p, b2)
    p2 = _sc_segsum(Y2, src, dst, n)
    Y3, y3r = _tc_mid(p2, y2r, inv, W3l, W3r, b3)
    p3 = _sc_segsum(Y3, src, dst, n)

    w_head = jnp.concatenate([Wreg, Wcls], axis=1)
    b_head = jnp.concatenate([breg, bcls])
    out = _tc_final(p3, y3r, inv, w_head, b_head)
    return out[:n, 0], out[:n, 1]


# speedup vs baseline: 6.5524x; 1.0072x over previous
import functools

import jax
import jax.numpy as jnp
from jax import lax
from jax.experimental import pallas as pl
from jax.experimental.pallas import tpu as pltpu
from jax.experimental.pallas import tpu_sc as plsc

_CHUNK = 128
_PAD_COLS = 16


def _node_rows(n):
    return ((n + 1 + 2047) // 2048) * 2048


def _sc_segsum(y, src, dst, n):
    R, D = y.shape
    E = src.shape[0]
    mesh = plsc.VectorSubcoreMesh(core_axis_name="c", subcore_axis_name="s")
    NC, NS = mesh.num_cores, mesh.num_subcores
    NW = NC * NS
    K = 4
    cpt = -(-E // (NW * _CHUNK * K)) * K
    E_pad = NW * cpt * _CHUNK
    G = cpt // K
    src = jnp.concatenate([src, jnp.zeros((E_pad - E,), jnp.int32)])
    dst = jnp.concatenate([dst, jnp.full((E_pad - E,), n, jnp.int32)])
    src_r = src.reshape(NW, cpt, _CHUNK)
    dst_r = dst.reshape(NW, cpt, _CHUNK)
    zeros = jnp.zeros((R, D), jnp.float32)
    rpt = R // NS

    @functools.partial(
        pl.kernel,
        out_type=jax.ShapeDtypeStruct((NC, R, D), jnp.float32),
        mesh=mesh,
        scratch_types=[
            pltpu.VMEM((cpt, _CHUNK), jnp.int32),
            pltpu.VMEM((cpt, _CHUNK), jnp.int32),
            pltpu.VMEM((K, _CHUNK, D), jnp.float32),
            pltpu.VMEM_SHARED((R, D), jnp.float32),
            pltpu.SemaphoreType.DMA,
            pltpu.SemaphoreType.DMA,
        ],
        compiler_params=pltpu.CompilerParams(use_tc_tiling_on_sc=False),
    )
    def seg_kernel(y_hbm, src_hbm, dst_hbm, z_hbm, out_hbm,
                   sidx, didx, rows, agg, sem, ssem):
        cid = lax.axis_index("c")
        sid = lax.axis_index("s")
        wid = cid * NS + sid
        base = sid * rpt
        pltpu.sync_copy(src_hbm.at[wid], sidx)
        pltpu.sync_copy(dst_hbm.at[wid], didx)
        pltpu.sync_copy(z_hbm.at[pl.ds(base, rpt)], agg.at[pl.ds(base, rpt)])
        plsc.subcore_barrier()

        def body(g, carry):
            c0 = g * K
            gathers = [
                pltpu.async_copy(y_hbm.at[sidx.at[c0 + b]], rows.at[b], sem)
                for b in range(K)
            ]
            for d in gathers:
                d.wait()
            scatters = [
                pltpu.async_copy(rows.at[b], agg.at[didx.at[c0 + b]],
                                 ssem, add=True)
                for b in range(K)
            ]
            for d in scatters:
                d.wait()
            return carry

        lax.fori_loop(0, G, body, 0)
        plsc.subcore_barrier()
        pltpu.sync_copy(agg.at[pl.ds(base, rpt)],
                        out_hbm.at[cid, pl.ds(base, rpt)])

    return seg_kernel(y, src_r, dst_r, zeros)


def _tc_proj_first(x, wl, ones_bias, wr, br):
    R = x.shape[0]
    D = wl.shape[1]

    def body(x_ref, wl_ref, ob_ref, wr_ref, br_ref, y_ref, yr_ref):
        xv = x_ref[...]
        y_ref[...] = (jnp.dot(xv, wl_ref[...],
                              preferred_element_type=jnp.float32)
                      + ob_ref[...][None, :])
        yr_ref[...] = (jnp.dot(xv, wr_ref[...],
                               preferred_element_type=jnp.float32)
                       + br_ref[...][None, :])

    return pl.pallas_call(
        body,
        out_shape=[jax.ShapeDtypeStruct((R, D), jnp.float32),
                   jax.ShapeDtypeStruct((R, D), jnp.float32)],
    )(x, wl, ones_bias, wr, br)


def _tc_mean_proj(p, yr, sel, wl, wr, br):
    _, R, _ = p.shape
    D2 = wl.shape[1]

    def body(p_ref, yr_ref, sel_ref, wl_ref, wr_ref, br_ref,
             y_ref, y2r_ref, inv_ref):
        agg = p_ref[0] + p_ref[1]
        cnt = jnp.dot(agg, sel_ref[...],
                      preferred_element_type=jnp.float32)
        inv = 1.0 / jnp.maximum(cnt, 1.0)
        h = jnp.maximum(agg * inv + yr_ref[...], 0.0)
        y_ref[...] = jnp.dot(h, wl_ref[...],
                             preferred_element_type=jnp.float32)
        y2r_ref[...] = (jnp.dot(h, wr_ref[...],
                                preferred_element_type=jnp.float32)
                        + br_ref[...][None, :])
        inv_ref[...] = inv

    return pl.pallas_call(
        body,
        out_shape=[jax.ShapeDtypeStruct((R, D2), jnp.float32),
                   jax.ShapeDtypeStruct((R, D2), jnp.float32),
                   jax.ShapeDtypeStruct((R, 1), jnp.float32)],
    )(p, yr, sel, wl, wr, br)


def _tc_mid(p, yr, inv, wl, wr, br):
    _, R, _ = p.shape
    D2 = wl.shape[1]

    def body(p_ref, yr_ref, inv_ref, wl_ref, wr_ref, br_ref, y_ref, yr2_ref):
        agg = p_ref[0] + p_ref[1]
        h = jnp.maximum(agg * inv_ref[...] + yr_ref[...], 0.0)
        y_ref[...] = jnp.dot(h, wl_ref[...],
                             preferred_element_type=jnp.float32)
        yr2_ref[...] = (jnp.dot(h, wr_ref[...],
                                preferred_element_type=jnp.float32)
                        + br_ref[...][None, :])

    return pl.pallas_call(
        body,
        out_shape=[jax.ShapeDtypeStruct((R, D2), jnp.float32),
                   jax.ShapeDtypeStruct((R, D2), jnp.float32)],
    )(p, yr, inv, wl, wr, br)


def _tc_final(p, yr, inv, w_head, b_head):
    _, R, _ = p.shape

    def body(p_ref, yr_ref, inv_ref, wh_ref, bh_ref, o_ref):
        agg = p_ref[0] + p_ref[1]
        h = jnp.maximum(agg * inv_ref[...] + yr_ref[...], 0.0)
        o_ref[...] = (jnp.dot(h, wh_ref[...],
                              preferred_element_type=jnp.float32)
                      + bh_ref[...][None, :])

    return pl.pallas_call(
        body,
        out_shape=jax.ShapeDtypeStruct((R, 2), jnp.float32),
    )(p, yr, inv, w_head, b_head)


def kernel(x, edge_index, W1l, W1r, b1, W2l, W2r, b2, W3l, W3r, b3,
           Wreg, breg, Wcls, bcls):
    n, d_in = x.shape
    R = _node_rows(n)
    d1 = W1l.shape[1]
    d1p = d1 + _PAD_COLS

    x_pad = jnp.zeros((R, d_in), jnp.float32).at[:n].set(x)
    src = edge_index[0].astype(jnp.int32)
    dst = edge_index[1].astype(jnp.int32)

    W1l_p = jnp.pad(W1l, ((0, 0), (0, _PAD_COLS)))
    ones_bias = jnp.concatenate(
        [jnp.zeros((d1,), jnp.float32), jnp.ones((_PAD_COLS,), jnp.float32)])
    W1r_p = jnp.pad(W1r, ((0, 0), (0, _PAD_COLS)))
    b1_p = jnp.pad(b1, (0, _PAD_COLS))
    sel = jnp.zeros((d1p, 1), jnp.float32).at[d1, 0].set(1.0)
    W2l_p = jnp.pad(W2l, ((0, _PAD_COLS), (0, 0)))
    W2r_p = jnp.pad(W2r, ((0, _PAD_COLS), (0, 0)))

    Y1, y1r = _tc_proj_first(x_pad, W1l_p, ones_bias, W1r_p, b1_p)
    p1 = _sc_segsum(Y1, src, dst, n)
    Y2, y2r, inv = _tc_mean_proj(p1, y1r, sel, W2l_p, W2r_p, b2)
    p2 = _sc_segsum(Y2, src, dst, n)
    Y3, y3r = _tc_mid(p2, y2r, inv, W3l, W3r, b3)
    p3 = _sc_segsum(Y3, src, dst, n)

    w_head = jnp.concatenate([Wreg, Wcls], axis=1)
    b_head = jnp.concatenate([breg, bcls])
    out = _tc_final(p3, y3r, inv, w_head, b_head)
    return out[:n, 0], out[:n, 1]


# speedup vs baseline: 6.5829x; 1.0046x over previous
import functools

import jax
import jax.numpy as jnp
from jax import lax
from jax.experimental import pallas as pl
from jax.experimental.pallas import tpu as pltpu
from jax.experimental.pallas import tpu_sc as plsc

_CHUNK = 128
_PAD_COLS = 16


def _node_rows(n):
    return ((n + 1 + 2047) // 2048) * 2048


def _sc_segsum(y, src, dst, n):
    R, D = y.shape
    E = src.shape[0]
    mesh = plsc.VectorSubcoreMesh(core_axis_name="c", subcore_axis_name="s")
    NC, NS = mesh.num_cores, mesh.num_subcores
    NW = NC * NS
    K = 4
    cpt = -(-E // (NW * _CHUNK * K)) * K
    E_pad = NW * cpt * _CHUNK
    G = cpt // K
    n_dummy = E_pad - E
    src = jnp.concatenate([src, jnp.zeros((n_dummy,), jnp.int32)])
    dst = jnp.concatenate(
        [dst, n + (jnp.arange(n_dummy, dtype=jnp.int32) % (R - n))])
    src_r = src.reshape(NW, cpt, _CHUNK)
    dst_r = dst.reshape(NW, cpt, _CHUNK)
    zeros = jnp.zeros((R, D), jnp.float32)
    rpt = R // NS

    @functools.partial(
        pl.kernel,
        out_type=jax.ShapeDtypeStruct((NC, R, D), jnp.float32),
        mesh=mesh,
        scratch_types=[
            pltpu.VMEM((cpt, _CHUNK), jnp.int32),
            pltpu.VMEM((cpt, _CHUNK), jnp.int32),
            pltpu.VMEM((K, _CHUNK, D), jnp.float32),
            pltpu.VMEM_SHARED((R, D), jnp.float32),
            pltpu.SemaphoreType.DMA,
            pltpu.SemaphoreType.DMA,
        ],
        compiler_params=pltpu.CompilerParams(use_tc_tiling_on_sc=False),
    )
    def seg_kernel(y_hbm, src_hbm, dst_hbm, z_hbm, out_hbm,
                   sidx, didx, rows, agg, sem, ssem):
        cid = lax.axis_index("c")
        sid = lax.axis_index("s")
        wid = cid * NS + sid
        base = sid * rpt
        pltpu.sync_copy(src_hbm.at[wid], sidx)
        pltpu.sync_copy(dst_hbm.at[wid], didx)
        pltpu.sync_copy(z_hbm.at[pl.ds(base, rpt)], agg.at[pl.ds(base, rpt)])
        plsc.subcore_barrier()

        def body(g, carry):
            c0 = g * K
            gathers = [
                pltpu.async_copy(y_hbm.at[sidx.at[c0 + b]], rows.at[b], sem)
                for b in range(K)
            ]
            for d in gathers:
                d.wait()
            scatters = [
                pltpu.async_copy(rows.at[b], agg.at[didx.at[c0 + b]],
                                 ssem, add=True)
                for b in range(K)
            ]
            for d in scatters:
                d.wait()
            return carry

        lax.fori_loop(0, G, body, 0)
        plsc.subcore_barrier()
        pltpu.sync_copy(agg.at[pl.ds(base, rpt)],
                        out_hbm.at[cid, pl.ds(base, rpt)])

    return seg_kernel(y, src_r, dst_r, zeros)


def _tc_proj_first(x, wl, ones_bias, wr, br):
    R = x.shape[0]
    D = wl.shape[1]

    def body(x_ref, wl_ref, ob_ref, wr_ref, br_ref, y_ref, yr_ref):
        xv = x_ref[...]
        y_ref[...] = (jnp.dot(xv, wl_ref[...],
                              preferred_element_type=jnp.float32)
                      + ob_ref[...][None, :])
        yr_ref[...] = (jnp.dot(xv, wr_ref[...],
                               preferred_element_type=jnp.float32)
                       + br_ref[...][None, :])

    return pl.pallas_call(
        body,
        out_shape=[jax.ShapeDtypeStruct((R, D), jnp.float32),
                   jax.ShapeDtypeStruct((R, D), jnp.float32)],
    )(x, wl, ones_bias, wr, br)


def _tc_mean_proj(p, yr, sel, wl, wr, br):
    _, R, _ = p.shape
    D2 = wl.shape[1]

    def body(p_ref, yr_ref, sel_ref, wl_ref, wr_ref, br_ref,
             y_ref, y2r_ref, inv_ref):
        agg = p_ref[0] + p_ref[1]
        cnt = jnp.dot(agg, sel_ref[...],
                      preferred_element_type=jnp.float32)
        inv = 1.0 / jnp.maximum(cnt, 1.0)
        h = jnp.maximum(agg * inv + yr_ref[...], 0.0)
        y_ref[...] = jnp.dot(h, wl_ref[...],
                             preferred_element_type=jnp.float32)
        y2r_ref[...] = (jnp.dot(h, wr_ref[...],
                                preferred_element_type=jnp.float32)
                        + br_ref[...][None, :])
        inv_ref[...] = inv

    return pl.pallas_call(
        body,
        out_shape=[jax.ShapeDtypeStruct((R, D2), jnp.float32),
                   jax.ShapeDtypeStruct((R, D2), jnp.float32),
                   jax.ShapeDtypeStruct((R, 1), jnp.float32)],
    )(p, yr, sel, wl, wr, br)


def _tc_mid(p, yr, inv, wl, wr, br):
    _, R, _ = p.shape
    D2 = wl.shape[1]

    def body(p_ref, yr_ref, inv_ref, wl_ref, wr_ref, br_ref, y_ref, yr2_ref):
        agg = p_ref[0] + p_ref[1]
        h = jnp.maximum(agg * inv_ref[...] + yr_ref[...], 0.0)
        y_ref[...] = jnp.dot(h, wl_ref[...],
                             preferred_element_type=jnp.float32)
        yr2_ref[...] = (jnp.dot(h, wr_ref[...],
                                preferred_element_type=jnp.float32)
                        + br_ref[...][None, :])

    return pl.pallas_call(
        body,
        out_shape=[jax.ShapeDtypeStruct((R, D2), jnp.float32),
                   jax.ShapeDtypeStruct((R, D2), jnp.float32)],
    )(p, yr, inv, wl, wr, br)


def _tc_final(p, yr, inv, w_head, b_head):
    _, R, _ = p.shape

    def body(p_ref, yr_ref, inv_ref, wh_ref, bh_ref, o_ref):
        agg = p_ref[0] + p_ref[1]
        h = jnp.maximum(agg * inv_ref[...] + yr_ref[...], 0.0)
        o_ref[...] = (jnp.dot(h, wh_ref[...],
                              preferred_element_type=jnp.float32)
                      + bh_ref[...][None, :])

    return pl.pallas_call(
        body,
        out_shape=jax.ShapeDtypeStruct((R, 2), jnp.float32),
    )(p, yr, inv, w_head, b_head)


def kernel(x, edge_index, W1l, W1r, b1, W2l, W2r, b2, W3l, W3r, b3,
           Wreg, breg, Wcls, bcls):
    n, d_in = x.shape
    R = _node_rows(n)
    d1 = W1l.shape[1]
    d1p = d1 + _PAD_COLS

    x_pad = jnp.zeros((R, d_in), jnp.float32).at[:n].set(x)
    src = edge_index[0].astype(jnp.int32)
    dst = edge_index[1].astype(jnp.int32)

    W1l_p = jnp.pad(W1l, ((0, 0), (0, _PAD_COLS)))
    ones_bias = jnp.concatenate(
        [jnp.zeros((d1,), jnp.float32), jnp.ones((_PAD_COLS,), jnp.float32)])
    W1r_p = jnp.pad(W1r, ((0, 0), (0, _PAD_COLS)))
    b1_p = jnp.pad(b1, (0, _PAD_COLS))
    sel = jnp.zeros((d1p, 1), jnp.float32).at[d1, 0].set(1.0)
    W2l_p = jnp.pad(W2l, ((0, _PAD_COLS), (0, 0)))
    W2r_p = jnp.pad(W2r, ((0, _PAD_COLS), (0, 0)))

    Y1, y1r = _tc_proj_first(x_pad, W1l_p, ones_bias, W1r_p, b1_p)
    p1 = _sc_segsum(Y1, src, dst, n)
    Y2, y2r, inv = _tc_mean_proj(p1, y1r, sel, W2l_p, W2r_p, b2)
    p2 = _sc_segsum(Y2, src, dst, n)
    Y3, y3r = _tc_mid(p2, y2r, inv, W3l, W3r, b3)
    p3 = _sc_segsum(Y3, src, dst, n)

    w_head = jnp.concatenate([Wreg, Wcls], axis=1)
    b_head = jnp.concatenate([breg, bcls])
    out = _tc_final(p3, y3r, inv, w_head, b_head)
    return out[:n, 0], out[:n, 1]


# speedup vs baseline: 7.6972x; 1.1693x over previous
import functools

import jax
import jax.numpy as jnp
from jax import lax
from jax.experimental import pallas as pl
from jax.experimental.pallas import tpu as pltpu
from jax.experimental.pallas import tpu_sc as plsc

_CHUNK = 128
_PAD_COLS = 16
_FRAC0 = 0.78


def _node_rows(n):
    return ((n + 1 + 2047) // 2048) * 2048


def _sc_segsum(y, src, dst, n):
    R, D = y.shape
    E = src.shape[0]
    mesh = plsc.VectorSubcoreMesh(core_axis_name="c", subcore_axis_name="s")
    NC, NS = mesh.num_cores, mesh.num_subcores
    K = 4
    C = -(-E // _CHUNK)
    cpt0 = -(-int(C * _FRAC0) // (NS * K)) * K
    cpt1 = -(-(C - NS * cpt0) // (NS * K)) * K
    E_pad = NS * (cpt0 + cpt1) * _CHUNK
    n_dummy = E_pad - E
    src = jnp.concatenate([src, jnp.zeros((n_dummy,), jnp.int32)])
    dst = jnp.concatenate(
        [dst, n + (jnp.arange(n_dummy, dtype=jnp.int32) % (R - n))])
    e0 = NS * cpt0 * _CHUNK
    src0 = src[:e0].reshape(NS, cpt0, _CHUNK)
    dst0 = dst[:e0].reshape(NS, cpt0, _CHUNK)
    src1 = src[e0:].reshape(NS, cpt1, _CHUNK)
    dst1 = dst[e0:].reshape(NS, cpt1, _CHUNK)
    zeros = jnp.zeros((R // NS, D), jnp.float32)
    rpt = R // NS

    @functools.partial(
        pl.kernel,
        out_type=jax.ShapeDtypeStruct((NC, R, D), jnp.float32),
        mesh=mesh,
        scratch_types=[
            pltpu.VMEM((cpt0, _CHUNK), jnp.int32),
            pltpu.VMEM((cpt0, _CHUNK), jnp.int32),
            pltpu.VMEM((K, _CHUNK, D), jnp.float32),
            pltpu.VMEM_SHARED((R, D), jnp.float32),
            pltpu.SemaphoreType.DMA,
            pltpu.SemaphoreType.DMA,
        ],
        compiler_params=pltpu.CompilerParams(use_tc_tiling_on_sc=False),
    )
    def seg_kernel(y_hbm, src0_hbm, dst0_hbm, src1_hbm, dst1_hbm, z_hbm,
                   out_hbm, sidx, didx, rows, agg, sem, ssem):
        cid = lax.axis_index("c")
        sid = lax.axis_index("s")
        base = sid * rpt

        @pl.when(cid == 0)
        def _():
            pltpu.sync_copy(src0_hbm.at[sid], sidx)
            pltpu.sync_copy(dst0_hbm.at[sid], didx)

        @pl.when(cid != 0)
        def _():
            pltpu.sync_copy(src1_hbm.at[sid], sidx.at[pl.ds(0, cpt1)])
            pltpu.sync_copy(dst1_hbm.at[sid], didx.at[pl.ds(0, cpt1)])

        pltpu.sync_copy(z_hbm, agg.at[pl.ds(base, rpt)])
        plsc.subcore_barrier()

        def body(g, carry):
            c0 = g * K
            gathers = [
                pltpu.async_copy(y_hbm.at[sidx.at[c0 + b]], rows.at[b], sem)
                for b in range(K)
            ]
            for d in gathers:
                d.wait()
            scatters = [
                pltpu.async_copy(rows.at[b], agg.at[didx.at[c0 + b]],
                                 ssem, add=True)
                for b in range(K)
            ]
            for d in scatters:
                d.wait()
            return carry

        n_groups = jnp.where(cid == 0, cpt0 // K, cpt1 // K)
        lax.fori_loop(0, n_groups, body, 0)
        plsc.subcore_barrier()
        pltpu.sync_copy(agg.at[pl.ds(base, rpt)],
                        out_hbm.at[cid, pl.ds(base, rpt)])

    return seg_kernel(y, src0, dst0, src1, dst1, zeros)


def _tc_proj_first(x, wl, ones_bias, wr, br):
    R = x.shape[0]
    D = wl.shape[1]

    def body(x_ref, wl_ref, ob_ref, wr_ref, br_ref, y_ref, yr_ref):
        xv = x_ref[...]
        y_ref[...] = (jnp.dot(xv, wl_ref[...],
                              preferred_element_type=jnp.float32)
                      + ob_ref[...][None, :])
        yr_ref[...] = (jnp.dot(xv, wr_ref[...],
                               preferred_element_type=jnp.float32)
                       + br_ref[...][None, :])

    return pl.pallas_call(
        body,
        out_shape=[jax.ShapeDtypeStruct((R, D), jnp.float32),
                   jax.ShapeDtypeStruct((R, D), jnp.float32)],
    )(x, wl, ones_bias, wr, br)


def _tc_mean_proj(p, yr, sel, wl, wr, br):
    _, R, _ = p.shape
    D2 = wl.shape[1]

    def body(p_ref, yr_ref, sel_ref, wl_ref, wr_ref, br_ref,
             y_ref, y2r_ref, inv_ref):
        agg = p_ref[0] + p_ref[1]
        cnt = jnp.dot(agg, sel_ref[...],
                      preferred_element_type=jnp.float32)
        inv = 1.0 / jnp.maximum(cnt, 1.0)
        h = jnp.maximum(agg * inv + yr_ref[...], 0.0)
        y_ref[...] = jnp.dot(h, wl_ref[...],
                             preferred_element_type=jnp.float32)
        y2r_ref[...] = (jnp.dot(h, wr_ref[...],
                                preferred_element_type=jnp.float32)
                        + br_ref[...][None, :])
        inv_ref[...] = inv

    return pl.pallas_call(
        body,
        out_shape=[jax.ShapeDtypeStruct((R, D2), jnp.float32),
                   jax.ShapeDtypeStruct((R, D2), jnp.float32),
                   jax.ShapeDtypeStruct((R, 1), jnp.float32)],
    )(p, yr, sel, wl, wr, br)


def _tc_mid(p, yr, inv, wl, wr, br):
    _, R, _ = p.shape
    D2 = wl.shape[1]

    def body(p_ref, yr_ref, inv_ref, wl_ref, wr_ref, br_ref, y_ref, yr2_ref):
        agg = p_ref[0] + p_ref[1]
        h = jnp.maximum(agg * inv_ref[...] + yr_ref[...], 0.0)
        y_ref[...] = jnp.dot(h, wl_ref[...],
                             preferred_element_type=jnp.float32)
        yr2_ref[...] = (jnp.dot(h, wr_ref[...],
                                preferred_element_type=jnp.float32)
                        + br_ref[...][None, :])

    return pl.pallas_call(
        body,
        out_shape=[jax.ShapeDtypeStruct((R, D2), jnp.float32),
                   jax.ShapeDtypeStruct((R, D2), jnp.float32)],
    )(p, yr, inv, wl, wr, br)


def _tc_final(p, yr, inv, w_head, b_head):
    _, R, _ = p.shape

    def body(p_ref, yr_ref, inv_ref, wh_ref, bh_ref, o_ref):
        agg = p_ref[0] + p_ref[1]
        h = jnp.maximum(agg * inv_ref[...] + yr_ref[...], 0.0)
        o_ref[...] = (jnp.dot(h, wh_ref[...],
                              preferred_element_type=jnp.float32)
                      + bh_ref[...][None, :])

    return pl.pallas_call(
        body,
        out_shape=jax.ShapeDtypeStruct((R, 2), jnp.float32),
    )(p, yr, inv, w_head, b_head)


def kernel(x, edge_index, W1l, W1r, b1, W2l, W2r, b2, W3l, W3r, b3,
           Wreg, breg, Wcls, bcls):
    n, d_in = x.shape
    R = _node_rows(n)
    d1 = W1l.shape[1]
    d1p = d1 + _PAD_COLS

    x_pad = jnp.zeros((R, d_in), jnp.float32).at[:n].set(x)
    src = edge_index[0].astype(jnp.int32)
    dst = edge_index[1].astype(jnp.int32)

    W1l_p = jnp.pad(W1l, ((0, 0), (0, _PAD_COLS)))
    ones_bias = jnp.concatenate(
        [jnp.zeros((d1,), jnp.float32), jnp.ones((_PAD_COLS,), jnp.float32)])
    W1r_p = jnp.pad(W1r, ((0, 0), (0, _PAD_COLS)))
    b1_p = jnp.pad(b1, (0, _PAD_COLS))
    sel = jnp.zeros((d1p, 1), jnp.float32).at[d1, 0].set(1.0)
    W2l_p = jnp.pad(W2l, ((0, _PAD_COLS), (0, 0)))
    W2r_p = jnp.pad(W2r, ((0, _PAD_COLS), (0, 0)))

    Y1, y1r = _tc_proj_first(x_pad, W1l_p, ones_bias, W1r_p, b1_p)
    p1 = _sc_segsum(Y1, src, dst, n)
    Y2, y2r, inv = _tc_mean_proj(p1, y1r, sel, W2l_p, W2r_p, b2)
    p2 = _sc_segsum(Y2, src, dst, n)
    Y3, y3r = _tc_mid(p2, y2r, inv, W3l, W3r, b3)
    p3 = _sc_segsum(Y3, src, dst, n)

    w_head = jnp.concatenate([Wreg, Wcls], axis=1)
    b_head = jnp.concatenate([breg, bcls])
    out = _tc_final(p3, y3r, inv, w_head, b_head)
    return out[:n, 0], out[:n, 1]
